# Initial kernel scaffold; baseline (speedup 1.0000x reference)
import jax
import jax.numpy as jnp
from jax.experimental import pallas as pl


def kernel(states, delta_u, W_ih, W_hh, b_ih, b_hh, speaker_ids):
    raise NotImplementedError("write your pallas kernel here")


# speedup vs baseline: 6.0783x; 6.0783x over previous
import jax
import jax.numpy as jnp
from jax.experimental import pallas as pl
from jax.experimental.pallas import tpu as pltpu

B = 16384
S = 16
D = 128
BB = 512


def _gru_block(states_ref, du_ref, wih_t_ref, whh_t_ref, b_ih_ref, b_hh_ref,
               ids_ref, out_ref):
    st = states_ref[...]
    ids = jnp.clip(ids_ref[...], 0, S - 1)

    h_old = jnp.zeros((BB, D), dtype=jnp.float32)
    for s in range(S):
        m = (ids == s).astype(jnp.float32)
        h_old = h_old + st[:, s * D:(s + 1) * D] * m

    gi = jnp.dot(du_ref[...], wih_t_ref[...],
                 preferred_element_type=jnp.float32) + b_ih_ref[0]
    gh = jnp.dot(h_old, whh_t_ref[...],
                 preferred_element_type=jnp.float32) + b_hh_ref[0]

    r = jax.nn.sigmoid(gi[:, :D] + gh[:, :D])
    z = jax.nn.sigmoid(gi[:, D:2 * D] + gh[:, D:2 * D])
    n = jnp.tanh(gi[:, 2 * D:] + r * gh[:, 2 * D:])
    h_new = (1.0 - z) * n + z * h_old

    for s in range(S):
        m = ids == s
        out_ref[:, s * D:(s + 1) * D] = jnp.where(
            m, h_new, st[:, s * D:(s + 1) * D])


def kernel(states, delta_u, W_ih, W_hh, b_ih, b_hh, speaker_ids):
    nb = B // BB
    st2 = states.reshape(B, S * D)
    ids2 = speaker_ids.astype(jnp.int32).reshape(B, 1)
    wih_t = W_ih.T
    whh_t = W_hh.T
    b_ih2 = b_ih.reshape(1, 3 * D)
    b_hh2 = b_hh.reshape(1, 3 * D)

    out = pl.pallas_call(
        _gru_block,
        grid=(nb,),
        in_specs=[
            pl.BlockSpec((BB, S * D), lambda i: (i, 0)),
            pl.BlockSpec((BB, D), lambda i: (i, 0)),
            pl.BlockSpec((D, 3 * D), lambda i: (0, 0)),
            pl.BlockSpec((D, 3 * D), lambda i: (0, 0)),
            pl.BlockSpec((1, 3 * D), lambda i: (0, 0)),
            pl.BlockSpec((1, 3 * D), lambda i: (0, 0)),
            pl.BlockSpec((BB, 1), lambda i: (i, 0)),
        ],
        out_specs=pl.BlockSpec((BB, S * D), lambda i: (i, 0)),
        out_shape=jax.ShapeDtypeStruct((B, S * D), jnp.float32),
        compiler_params=pltpu.CompilerParams(
            dimension_semantics=("arbitrary",),
        ),
    )(st2, delta_u, wih_t, whh_t, b_ih2, b_hh2, ids2)
    return out.reshape(B, S, D)


# speedup vs baseline: 6.1827x; 1.0172x over previous
import jax
import jax.numpy as jnp
from jax.experimental import pallas as pl
from jax.experimental.pallas import tpu as pltpu

B = 16384
S = 16
D = 128
BB = 1024


def _gru_block(states_ref, du_ref, wih_t_ref, whh_t_ref, b_ih_ref, b_hh_ref,
               ids_ref, out_ref):
    st = states_ref[...]
    ids = jnp.clip(ids_ref[...], 0, S - 1)

    h_old = jnp.zeros((BB, D), dtype=jnp.float32)
    for s in range(S):
        m = (ids == s).astype(jnp.float32)
        h_old = h_old + st[:, s * D:(s + 1) * D] * m

    gi = jnp.dot(du_ref[...], wih_t_ref[...],
                 preferred_element_type=jnp.float32) + b_ih_ref[0]
    gh = jnp.dot(h_old, whh_t_ref[...],
                 preferred_element_type=jnp.float32) + b_hh_ref[0]

    r = jax.nn.sigmoid(gi[:, :D] + gh[:, :D])
    z = jax.nn.sigmoid(gi[:, D:2 * D] + gh[:, D:2 * D])
    n = jnp.tanh(gi[:, 2 * D:] + r * gh[:, 2 * D:])
    h_new = (1.0 - z) * n + z * h_old

    for s in range(S):
        m = ids == s
        out_ref[:, s * D:(s + 1) * D] = jnp.where(
            m, h_new, st[:, s * D:(s + 1) * D])


def kernel(states, delta_u, W_ih, W_hh, b_ih, b_hh, speaker_ids):
    nb = B // BB
    st2 = states.reshape(B, S * D)
    ids2 = speaker_ids.astype(jnp.int32).reshape(B, 1)
    wih_t = W_ih.T
    whh_t = W_hh.T
    b_ih2 = b_ih.reshape(1, 3 * D)
    b_hh2 = b_hh.reshape(1, 3 * D)

    out = pl.pallas_call(
        _gru_block,
        grid=(nb,),
        in_specs=[
            pl.BlockSpec((BB, S * D), lambda i: (i, 0)),
            pl.BlockSpec((BB, D), lambda i: (i, 0)),
            pl.BlockSpec((D, 3 * D), lambda i: (0, 0)),
            pl.BlockSpec((D, 3 * D), lambda i: (0, 0)),
            pl.BlockSpec((1, 3 * D), lambda i: (0, 0)),
            pl.BlockSpec((1, 3 * D), lambda i: (0, 0)),
            pl.BlockSpec((BB, 1), lambda i: (i, 0)),
        ],
        out_specs=pl.BlockSpec((BB, S * D), lambda i: (i, 0)),
        out_shape=jax.ShapeDtypeStruct((B, S * D), jnp.float32),
        compiler_params=pltpu.CompilerParams(
            dimension_semantics=("arbitrary",),
        ),
    )(st2, delta_u, wih_t, whh_t, b_ih2, b_hh2, ids2)
    return out.reshape(B, S, D)
